# Initial kernel scaffold; baseline (speedup 1.0000x reference)
import jax
import jax.numpy as jnp
from jax.experimental import pallas as pl


def kernel(x, W1, b1, W2, b2):
    raise NotImplementedError("write your pallas kernel here")


# speedup vs baseline: 7.0368x; 7.0368x over previous
import functools

import jax
import jax.numpy as jnp
from jax.experimental import pallas as pl
from jax.experimental.pallas import tpu as pltpu

B, D = 128, 8192
H = D // 4
MIN_S, MAX_S = 0.05, 0.3
KBLK = 1024
NSTEPS = D // KBLK


def _fused_kernel(x_ref, w1_ref, b1_ref, w2_ref, b2_ref,
                  sparse_ref, mask_ref, s_ref, act_ref, l1_ref, acc_ref):
    j = pl.program_id(0)

    @pl.when(j == 0)
    def _init():
        acc_ref[...] = jnp.zeros_like(acc_ref)

    xblk = x_ref[:, pl.ds(j * KBLK, KBLK)]
    acc_ref[...] += jnp.dot(xblk, w1_ref[...],
                            preferred_element_type=jnp.float32)

    @pl.when(j == NSTEPS - 1)
    def _finish():
        h = jnp.maximum(acc_ref[...] + b1_ref[...], 0.0)
        t = jnp.dot(h, w2_ref[...], preferred_element_type=jnp.float32)
        s = jax.nn.sigmoid(t + b2_ref[...])
        sparsity = MIN_S + (MAX_S - MIN_S) * s
        k = jnp.clip(jnp.round(D * (1.0 - sparsity)).astype(jnp.int32), 1, D)

        x = x_ref[...]
        bits = jax.lax.bitcast_convert_type(x, jnp.int32) & jnp.int32(0x7FFFFFFF)

        def body(i, p):
            bit = jnp.int32(30) - i
            c = p | (jnp.int32(1) << bit)
            cnt = jnp.sum((bits < c).astype(jnp.int32), axis=1, keepdims=True)
            return jnp.where(cnt < k, c, p)

        p = jax.lax.fori_loop(0, 31, body, jnp.zeros((B, 1), jnp.int32))

        mask = (bits > p).astype(jnp.float32)
        sparse_x = x * mask
        s_ref[...] = sparsity
        mask_ref[...] = mask
        sparse_ref[...] = sparse_x
        nnz = jnp.sum(mask, axis=1, keepdims=True)
        act_ref[...] = nnz * (1.0 / D)
        l1_ref[...] = jnp.sum(jnp.abs(sparse_x), keepdims=True).reshape(1, 1) * (1.0 / B)


@jax.jit
def kernel(x, W1, b1, W2, b2):
    b1r = b1.reshape(1, H)
    b2r = b2.reshape(1, 1)
    out_shapes = (
        jax.ShapeDtypeStruct((B, D), jnp.float32),
        jax.ShapeDtypeStruct((B, D), jnp.float32),
        jax.ShapeDtypeStruct((B, 1), jnp.float32),
        jax.ShapeDtypeStruct((B, 1), jnp.float32),
        jax.ShapeDtypeStruct((1, 1), jnp.float32),
    )
    grid = (NSTEPS,)
    sparse_x, mask, sparsity, act, l1 = pl.pallas_call(
        _fused_kernel,
        grid=grid,
        in_specs=[
            pl.BlockSpec((B, D), lambda j: (0, 0)),
            pl.BlockSpec((KBLK, H), lambda j: (j, 0)),
            pl.BlockSpec((1, H), lambda j: (0, 0)),
            pl.BlockSpec((H, 1), lambda j: (0, 0)),
            pl.BlockSpec((1, 1), lambda j: (0, 0)),
        ],
        out_specs=(
            pl.BlockSpec((B, D), lambda j: (0, 0)),
            pl.BlockSpec((B, D), lambda j: (0, 0)),
            pl.BlockSpec((B, 1), lambda j: (0, 0)),
            pl.BlockSpec((B, 1), lambda j: (0, 0)),
            pl.BlockSpec((1, 1), lambda j: (0, 0)),
        ),
        out_shape=out_shapes,
        scratch_shapes=[pltpu.VMEM((B, H), jnp.float32)],
    )(x, W1, b1r, W2, b2r)
    return (sparse_x, mask, sparsity, act.reshape(B), l1.reshape(()))
